# baseline (speedup 1.0000x reference)
import functools

import jax
import jax.numpy as jnp
from jax import lax
from jax.experimental import pallas as pl
from jax.experimental.pallas import tpu as pltpu
from jax.experimental.pallas import tpu_sc as plsc

COL = 64
NC = 2
NS = 16
NW = NC * NS


def _make_gather(batch: int, seq: int, half: int):
    mesh = plsc.VectorSubcoreMesh(core_axis_name="c", subcore_axis_name="s")
    b_per_w = batch // NW

    @functools.partial(
        pl.kernel,
        mesh=mesh,
        out_type=jax.ShapeDtypeStruct((batch, seq, COL), jnp.float32),
        scratch_types=[
            pltpu.VMEM((seq,), jnp.int32),
            pltpu.VMEM((seq, COL), jnp.float32),
            pltpu.SemaphoreType.DMA,
        ],
        compiler_params=pltpu.CompilerParams(use_tc_tiling_on_sc=False),
    )
    def k(x_hbm, table_hbm, out_hbm, idx_v, rows_v, sem):
        wid = lax.axis_index("s") * NC + lax.axis_index("c")
        b0 = wid * b_per_w

        def body(i, carry):
            b = b0 + i
            pltpu.sync_copy(x_hbm.at[b], idx_v)
            cp0 = pltpu.async_copy(
                table_hbm.at[idx_v.at[pl.ds(0, half)]],
                rows_v.at[pl.ds(0, half)], sem)
            cp1 = pltpu.async_copy(
                table_hbm.at[idx_v.at[pl.ds(half, seq - half)]],
                rows_v.at[pl.ds(half, seq - half)], sem)
            cp0.wait()
            cp1.wait()
            pltpu.sync_copy(rows_v, out_hbm.at[b])
            return carry

        lax.fori_loop(0, b_per_w, body, 0)

    return k


def kernel(x, embed_weight):
    batch, seq = x.shape
    out = _make_gather(batch, seq, 104)(x.astype(jnp.int32), embed_weight)
    return out
